# Initial kernel scaffold; baseline (speedup 1.0000x reference)
import jax
import jax.numpy as jnp
from jax.experimental import pallas as pl


def kernel(embeds, dists_array, anchorset_id, edge_index, Wh, bh, Wq, Wk, Wv):
    raise NotImplementedError("write your pallas kernel here")


# speedup vs baseline: 3.7881x; 3.7881x over previous
import functools

import jax
import jax.numpy as jnp
import numpy as np
from jax import lax
from jax.experimental import pallas as pl
from jax.experimental.pallas import tpu as pltpu
from jax.experimental.pallas import tpu_sc as plsc

LATDIM = 128
ANCHOR = 32
N_NODES = 10000
N_EDGES = 320000
HEADS = 4
DHEAD = 32
NBLK = 625
E_TOT = N_EDGES + 2 * int(N_EDGES * 0.1) + N_NODES

NW = 32
C = 128
GROUPS = C // 16
CPW = 97
EPAD = NW * CPW * C
TOT_CHUNKS = NW * CPW
INV_SQRT_DH = 0.17677669529663687

_P = np.zeros((NBLK, NBLK), np.float32)
_j = np.arange(NBLK)
_P[_j, (2 * _j) % NBLK] += 1.0
_P[_j, (2 * _j + 1) % NBLK] += 1.0


def _dot(a, b):
    return jax.lax.dot(a, b, precision=jax.lax.Precision.HIGHEST,
                       preferred_element_type=jnp.float32)


def _dense_body(eview_ref, distsT_ref, s_ref, wh1_ref, wh2_ref, wq_ref,
                wk_ref, bh_ref, perm_ref, qa_ref, ka_ref, qb_ref, kb_ref):
    ev = eview_ref[...]
    h = ev[:, 0:LATDIM]
    for a in range(1, 16):
        h = h + ev[:, a * LATDIM:(a + 1) * LATDIM]
    t = _dot(s_ref[...], wh1_ref[...])
    p1 = _dot(distsT_ref[...], t)
    g = _dot(_dot(perm_ref[...], h), wh2_ref[...])
    pos_b = g * (1.0 / ANCHOR) + bh_ref[...]
    pos_a = p1 * (1.0 / ANCHOR)
    qa_ref[...] = _dot(pos_a, wq_ref[...])
    ka_ref[...] = _dot(pos_a, wk_ref[...])
    qb_ref[...] = _dot(pos_b, wq_ref[...])
    kb_ref[...] = _dot(pos_b, wk_ref[...])


_dense_call = pl.pallas_call(
    _dense_body,
    out_shape=[
        jax.ShapeDtypeStruct((N_NODES, LATDIM), jnp.float32),
        jax.ShapeDtypeStruct((N_NODES, LATDIM), jnp.float32),
        jax.ShapeDtypeStruct((NBLK, LATDIM), jnp.float32),
        jax.ShapeDtypeStruct((NBLK, LATDIM), jnp.float32),
    ],
)

_sc_mesh = plsc.VectorSubcoreMesh(core_axis_name="c", subcore_axis_name="s")


@functools.partial(
    pl.kernel, mesh=_sc_mesh,
    out_type=[
        jax.ShapeDtypeStruct((TOT_CHUNKS, HEADS, C), jnp.float32),
        jax.ShapeDtypeStruct((2 * HEADS, N_NODES), jnp.float32),
    ],
    scratch_types=[
        pltpu.VMEM((C,), jnp.int32),
        pltpu.VMEM((C,), jnp.int32),
        pltpu.VMEM((C, LATDIM), jnp.float32),
        pltpu.VMEM((C, LATDIM), jnp.float32),
        pltpu.VMEM((HEADS, C), jnp.float32),
        pltpu.VMEM((2000,), jnp.float32),
        pltpu.VMEM_SHARED((N_NODES,), jnp.float32),
        pltpu.VMEM_SHARED((N_NODES,), jnp.float32),
        pltpu.VMEM_SHARED((N_NODES,), jnp.float32),
        pltpu.VMEM_SHARED((N_NODES,), jnp.float32),
        pltpu.SemaphoreType.DMA,
        pltpu.SemaphoreType.DMA,
    ],
    compiler_params=pltpu.CompilerParams(needs_layout_passes=False),
)
def _edge_kernel(q_hbm, k_hbm, rows_hbm, cols_hbm, expl_hbm, den_hbm,
                 rowidx, colidx, qrows, krows, explb, zbuf,
                 den0, den1, den2, den3, sem0, sem1):
    cid = lax.axis_index("c")
    sid = lax.axis_index("s")
    wid = sid * 2 + cid
    dens = [den0, den1, den2, den3]

    @pl.when(sid == 0)
    def _init():
        def zb(i, carry):
            zbuf[pl.ds(i * 16, 16)] = jnp.zeros((16,), jnp.float32)
            return carry
        lax.fori_loop(0, 125, zb, 0)
        for dh in dens:
            for tt in range(5):
                pltpu.sync_copy(zbuf, dh.at[pl.ds(tt * 2000, 2000)])

    plsc.subcore_barrier()

    def chunk_body(c, carry):
        gchunk = wid * CPW + c
        ebase = gchunk * C
        pltpu.sync_copy(rows_hbm.at[pl.ds(ebase, C)], rowidx)
        pltpu.sync_copy(cols_hbm.at[pl.ds(ebase, C)], colidx)
        cp_q = pltpu.async_copy(q_hbm.at[rowidx], qrows, sem0)
        cp_k = pltpu.async_copy(k_hbm.at[colidx], krows, sem1)
        cp_q.wait()
        cp_k.wait()

        def group_body(g, gcarry):
            lbase = g * 16
            ev = lax.broadcasted_iota(jnp.int32, (16,), 0) + lbase
            accs = [jnp.zeros((16,), jnp.float32) for _ in range(HEADS)]
            for dd in range(LATDIM):
                dv = jnp.full((16,), dd, jnp.int32)
                qv = plsc.load_gather(qrows, [ev, dv])
                kv = plsc.load_gather(krows, [ev, dv])
                accs[dd // DHEAD] = accs[dd // DHEAD] + qv * kv
            gid = ev + ebase
            maskf = jnp.where(gid < E_TOT, 1.0, 0.0).astype(jnp.float32)
            for hh in range(HEADS):
                lg = accs[hh] * INV_SQRT_DH
                lg = jnp.minimum(jnp.maximum(lg, -10.0), 10.0)
                explb[hh, pl.ds(lbase, 16)] = jnp.exp(lg) * maskf
            return gcarry

        lax.fori_loop(0, GROUPS, group_body, 0)
        pltpu.sync_copy(explb, expl_hbm.at[gchunk])
        for hh in range(HEADS):
            pltpu.sync_copy(explb.at[hh], dens[hh].at[rowidx], add=True)
        return carry

    lax.fori_loop(0, CPW, chunk_body, 0)
    plsc.subcore_barrier()

    @pl.when(sid == 0)
    def _flush():
        for hh in range(HEADS):
            pltpu.sync_copy(dens[hh], den_hbm.at[cid * HEADS + hh])


@functools.partial(
    pl.kernel, mesh=_sc_mesh,
    out_type=jax.ShapeDtypeStruct((EPAD,), jnp.float32),
    scratch_types=[
        pltpu.VMEM((C,), jnp.int32),
        pltpu.VMEM((HEADS, C), jnp.float32),
        pltpu.VMEM((C,), jnp.float32),
        pltpu.VMEM((C,), jnp.float32),
        pltpu.VMEM((C,), jnp.float32),
        pltpu.SemaphoreType.DMA,
    ],
    compiler_params=pltpu.CompilerParams(needs_layout_passes=False),
)
def _norm_kernel(expl_hbm, rows_hbm, d00, d01, d02, d03, d10, d11, d12, d13,
                 att_hbm, rowidx, explb, b0, b1, attb, sem0):
    cid = lax.axis_index("c")
    sid = lax.axis_index("s")
    wid = sid * 2 + cid
    dpart0 = [d00, d01, d02, d03]
    dpart1 = [d10, d11, d12, d13]

    def chunk_body(c, carry):
        gchunk = wid * CPW + c
        ebase = gchunk * C
        pltpu.sync_copy(rows_hbm.at[pl.ds(ebase, C)], rowidx)
        pltpu.sync_copy(expl_hbm.at[gchunk], explb)
        for hh in range(HEADS):
            pltpu.async_copy(dpart0[hh].at[rowidx], b0, sem0).wait()
            pltpu.async_copy(dpart1[hh].at[rowidx], b1, sem0).wait()

            def group_body(g, gcarry):
                sl = pl.ds(g * 16, 16)
                s = b0[sl] + b1[sl] + 1e-8
                e = explb[hh, sl]
                if hh == 0:
                    attb[sl] = e / s
                else:
                    attb[sl] = attb[sl] + e / s
                return gcarry

            lax.fori_loop(0, GROUPS, group_body, 0)
        pltpu.sync_copy(attb, att_hbm.at[pl.ds(ebase, C)])
        return carry

    lax.fori_loop(0, CPW, chunk_body, 0)


def kernel(embeds, dists_array, anchorset_id, edge_index, Wh, bh, Wq, Wk, Wv):
    embeds = embeds.astype(jnp.float32)
    eview = embeds.reshape(NBLK, 16 * LATDIM)
    dists_t = dists_array.T.astype(jnp.float32)
    s = embeds[anchorset_id]
    wh1 = Wh[:LATDIM]
    wh2 = Wh[LATDIM:]
    bh_row = bh.reshape(1, LATDIM)

    qa, ka, qb, kb = _dense_call(eview, dists_t, s, wh1, wh2, Wq, Wk,
                                 bh_row, jnp.asarray(_P))
    q = qa + jnp.tile(qb, (16, 1))
    k = ka + jnp.tile(kb, (16, 1))

    rows = edge_index[0]
    cols = edge_index[1]
    n_add = int(N_EDGES * 0.1)
    akey = jax.random.key(42)
    k1, k2 = jax.random.split(akey)
    idx1 = jax.random.randint(k1, (n_add,), 0, N_EDGES)
    idx2 = jax.random.randint(k2, (n_add,), 0, N_EDGES)
    add_rows = rows[idx1]
    add_cols = cols[idx2]
    loops = jnp.arange(N_NODES, dtype=rows.dtype)
    new_rows = jnp.concatenate([add_rows, add_cols, loops, rows])
    new_cols = jnp.concatenate([add_cols, add_rows, loops, cols])

    pad = jnp.zeros((EPAD - E_TOT,), new_rows.dtype)
    rpad = jnp.concatenate([new_rows, pad]).astype(jnp.int32)
    cpad = jnp.concatenate([new_cols, pad]).astype(jnp.int32)

    expl, den = _edge_kernel(q, k, rpad, cpad)
    att_pad = _norm_kernel(expl, rpad, den[0], den[1], den[2], den[3],
                           den[4], den[5], den[6], den[7])
    att_edge = att_pad[:E_TOT]
    new_edge_index = jnp.stack([new_rows, new_cols])
    return att_edge, new_edge_index


# speedup vs baseline: 4.9909x; 1.3175x over previous
import functools

import jax
import jax.numpy as jnp
import numpy as np
from jax import lax
from jax.experimental import pallas as pl
from jax.experimental.pallas import tpu as pltpu
from jax.experimental.pallas import tpu_sc as plsc

LATDIM = 128
ANCHOR = 32
N_NODES = 10000
N_EDGES = 320000
HEADS = 4
DHEAD = 32
NBLK = 625
E_TOT = N_EDGES + 2 * int(N_EDGES * 0.1) + N_NODES

NPAD = 10240
NW = 32
C = 128
GROUPS = C // 16
CPW = 97
EPAD = NW * CPW * C
TOT_CHUNKS = NW * CPW
INV_SQRT_DH = 0.17677669529663687

_P = np.zeros((NBLK, NBLK), np.float32)
_j = np.arange(NBLK)
_P[_j, (2 * _j) % NBLK] += 1.0
_P[_j, (2 * _j + 1) % NBLK] += 1.0


def _dot(a, b):
    return jax.lax.dot(a, b, precision=jax.lax.Precision.HIGHEST,
                       preferred_element_type=jnp.float32)


def _dense_body(eview_ref, distsT_ref, s_ref, wh1_ref, wh2_ref, wq_ref,
                wk_ref, bh_ref, perm_ref, qa_ref, ka_ref, qb_ref, kb_ref):
    ev = eview_ref[...]
    h = ev[:, 0:LATDIM]
    for a in range(1, 16):
        h = h + ev[:, a * LATDIM:(a + 1) * LATDIM]
    t = _dot(s_ref[...], wh1_ref[...])
    p1 = _dot(distsT_ref[...], t)
    g = _dot(_dot(perm_ref[...], h), wh2_ref[...])
    pos_b = g * (1.0 / ANCHOR) + bh_ref[...]
    pos_a = p1 * (1.0 / ANCHOR)
    qa_ref[...] = _dot(pos_a, wq_ref[...])
    ka_ref[...] = _dot(pos_a, wk_ref[...])
    qb_ref[...] = _dot(pos_b, wq_ref[...])
    kb_ref[...] = _dot(pos_b, wk_ref[...])


_dense_call = pl.pallas_call(
    _dense_body,
    out_shape=[
        jax.ShapeDtypeStruct((N_NODES, LATDIM), jnp.float32),
        jax.ShapeDtypeStruct((N_NODES, LATDIM), jnp.float32),
        jax.ShapeDtypeStruct((NBLK, LATDIM), jnp.float32),
        jax.ShapeDtypeStruct((NBLK, LATDIM), jnp.float32),
    ],
)

_sc_mesh = plsc.VectorSubcoreMesh(core_axis_name="c", subcore_axis_name="s")


@functools.partial(
    pl.kernel, mesh=_sc_mesh,
    out_type=[
        jax.ShapeDtypeStruct((TOT_CHUNKS, HEADS, C), jnp.float32),
        jax.ShapeDtypeStruct((2 * HEADS, NPAD), jnp.float32),
    ],
    scratch_types=[
        pltpu.VMEM((C,), jnp.int32),
        pltpu.VMEM((C,), jnp.int32),
        pltpu.VMEM((C, LATDIM), jnp.float32),
        pltpu.VMEM((C, LATDIM), jnp.float32),
        pltpu.VMEM((HEADS, C), jnp.float32),
        pltpu.VMEM((2048,), jnp.float32),
        pltpu.VMEM_SHARED((NPAD,), jnp.float32),
        pltpu.VMEM_SHARED((NPAD,), jnp.float32),
        pltpu.VMEM_SHARED((NPAD,), jnp.float32),
        pltpu.VMEM_SHARED((NPAD,), jnp.float32),
        pltpu.SemaphoreType.DMA,
        pltpu.SemaphoreType.DMA,
    ],
    compiler_params=pltpu.CompilerParams(needs_layout_passes=False),
)
def _edge_kernel(q_hbm, k_hbm, rows_hbm, cols_hbm, expl_hbm, den_hbm,
                 rowidx, colidx, qrows, krows, explb, zbuf,
                 den0, den1, den2, den3, sem0, sem1):
    cid = lax.axis_index("c")
    sid = lax.axis_index("s")
    wid = sid * 2 + cid
    dens = [den0, den1, den2, den3]

    @pl.when(sid == 0)
    def _init():
        def zb(i, carry):
            zbuf[pl.ds(i * 16, 16)] = jnp.zeros((16,), jnp.float32)
            return carry
        lax.fori_loop(0, 128, zb, 0)
        for dh in dens:
            for tt in range(5):
                pltpu.sync_copy(zbuf, dh.at[pl.ds(tt * 2048, 2048)])

    plsc.subcore_barrier()

    def chunk_body(c, carry):
        gchunk = wid * CPW + c
        ebase = gchunk * C
        pltpu.sync_copy(rows_hbm.at[pl.ds(ebase, C)], rowidx)
        pltpu.sync_copy(cols_hbm.at[pl.ds(ebase, C)], colidx)
        cp_q = pltpu.async_copy(q_hbm.at[rowidx], qrows, sem0)
        cp_k = pltpu.async_copy(k_hbm.at[colidx], krows, sem1)
        cp_q.wait()
        cp_k.wait()

        def group_body(g, gcarry):
            lbase = g * 16
            ev = lax.broadcasted_iota(jnp.int32, (16,), 0) + lbase
            accs = [jnp.zeros((16,), jnp.float32) for _ in range(HEADS)]
            for dd in range(LATDIM):
                dv = jnp.full((16,), dd, jnp.int32)
                qv = plsc.load_gather(qrows, [ev, dv])
                kv = plsc.load_gather(krows, [ev, dv])
                accs[dd // DHEAD] = accs[dd // DHEAD] + qv * kv
            gid = ev + ebase
            maskf = jnp.where(gid < E_TOT, 1.0, 0.0).astype(jnp.float32)
            for hh in range(HEADS):
                lg = accs[hh] * INV_SQRT_DH
                lg = jnp.minimum(jnp.maximum(lg, -10.0), 10.0)
                explb[hh, pl.ds(lbase, 16)] = jnp.exp(lg) * maskf
            return gcarry

        lax.fori_loop(0, GROUPS, group_body, 0)
        pltpu.sync_copy(explb, expl_hbm.at[gchunk])
        for hh in range(HEADS):
            pltpu.sync_copy(explb.at[hh], dens[hh].at[rowidx], add=True)
        return carry

    lax.fori_loop(0, CPW, chunk_body, 0)
    plsc.subcore_barrier()

    @pl.when(sid == 0)
    def _flush():
        for hh in range(HEADS):
            pltpu.sync_copy(dens[hh], den_hbm.at[cid * HEADS + hh])


@functools.partial(
    pl.kernel, mesh=_sc_mesh,
    out_type=jax.ShapeDtypeStruct((EPAD,), jnp.float32),
    scratch_types=[
        pltpu.VMEM((C,), jnp.int32),
        pltpu.VMEM((HEADS, C), jnp.float32),
        pltpu.VMEM((C,), jnp.float32),
        pltpu.VMEM((C,), jnp.float32),
        pltpu.VMEM((640,), jnp.float32),
        pltpu.VMEM((640,), jnp.float32),
        pltpu.VMEM_SHARED((NPAD,), jnp.float32),
        pltpu.VMEM_SHARED((NPAD,), jnp.float32),
        pltpu.VMEM_SHARED((NPAD,), jnp.float32),
        pltpu.VMEM_SHARED((NPAD,), jnp.float32),
        pltpu.SemaphoreType.DMA,
    ],
    compiler_params=pltpu.CompilerParams(needs_layout_passes=False),
)
def _norm_kernel(expl_hbm, rows_hbm, den_hbm, att_hbm,
                 rowidx, explb, db, attb, sa, sb,
                 sd0, sd1, sd2, sd3, sem0):
    cid = lax.axis_index("c")
    sid = lax.axis_index("s")
    wid = sid * 2 + cid
    sds = [sd0, sd1, sd2, sd3]
    quarter = sid // HEADS

    for hh in range(HEADS):
        @pl.when(sid % HEADS == hh)
        def _build(hh=hh):
            def bchunk(t, carry):
                off = quarter * 2560 + t * 640
                pltpu.sync_copy(den_hbm.at[hh, pl.ds(off, 640)], sa)
                pltpu.sync_copy(den_hbm.at[HEADS + hh, pl.ds(off, 640)], sb)

                def vec(i, c2):
                    sl = pl.ds(i * 16, 16)
                    sa[sl] = sa[sl] + sb[sl] + 1e-8
                    return c2

                lax.fori_loop(0, 40, vec, 0)
                pltpu.sync_copy(sa, sds[hh].at[pl.ds(off, 640)])
                return carry

            lax.fori_loop(0, 4, bchunk, 0)

    plsc.subcore_barrier()

    def chunk_body(c, carry):
        gchunk = wid * CPW + c
        ebase = gchunk * C
        pltpu.sync_copy(rows_hbm.at[pl.ds(ebase, C)], rowidx)
        pltpu.sync_copy(expl_hbm.at[gchunk], explb)
        for hh in range(HEADS):
            pltpu.async_copy(sds[hh].at[rowidx], db, sem0).wait()

            def group_body(g, gcarry, hh=hh):
                sl = pl.ds(g * 16, 16)
                e = explb[hh, sl]
                if hh == 0:
                    attb[sl] = e / db[sl]
                else:
                    attb[sl] = attb[sl] + e / db[sl]
                return gcarry

            lax.fori_loop(0, GROUPS, group_body, 0)
        pltpu.sync_copy(attb, att_hbm.at[pl.ds(ebase, C)])
        return carry

    lax.fori_loop(0, CPW, chunk_body, 0)


def kernel(embeds, dists_array, anchorset_id, edge_index, Wh, bh, Wq, Wk, Wv):
    embeds = embeds.astype(jnp.float32)
    eview = embeds.reshape(NBLK, 16 * LATDIM)
    dists_t = dists_array.T.astype(jnp.float32)
    s = embeds[anchorset_id]
    wh1 = Wh[:LATDIM]
    wh2 = Wh[LATDIM:]
    bh_row = bh.reshape(1, LATDIM)

    qa, ka, qb, kb = _dense_call(eview, dists_t, s, wh1, wh2, Wq, Wk,
                                 bh_row, jnp.asarray(_P))
    q = qa + jnp.tile(qb, (16, 1))
    k = ka + jnp.tile(kb, (16, 1))

    rows = edge_index[0]
    cols = edge_index[1]
    n_add = int(N_EDGES * 0.1)
    akey = jax.random.key(42)
    k1, k2 = jax.random.split(akey)
    idx1 = jax.random.randint(k1, (n_add,), 0, N_EDGES)
    idx2 = jax.random.randint(k2, (n_add,), 0, N_EDGES)
    add_rows = rows[idx1]
    add_cols = cols[idx2]
    loops = jnp.arange(N_NODES, dtype=rows.dtype)
    new_rows = jnp.concatenate([add_rows, add_cols, loops, rows])
    new_cols = jnp.concatenate([add_cols, add_rows, loops, cols])

    pad = jnp.zeros((EPAD - E_TOT,), new_rows.dtype)
    rpad = jnp.concatenate([new_rows, pad]).astype(jnp.int32)
    cpad = jnp.concatenate([new_cols, pad]).astype(jnp.int32)

    expl, den = _edge_kernel(q, k, rpad, cpad)
    att_pad = _norm_kernel(expl, rpad, den)
    att_edge = att_pad[:E_TOT]
    new_edge_index = jnp.stack([new_rows, new_cols])
    return att_edge, new_edge_index


# speedup vs baseline: 5.2184x; 1.0456x over previous
import functools

import jax
import jax.numpy as jnp
import numpy as np
from jax import lax
from jax.experimental import pallas as pl
from jax.experimental.pallas import tpu as pltpu
from jax.experimental.pallas import tpu_sc as plsc

LATDIM = 128
ANCHOR = 32
N_NODES = 10000
N_EDGES = 320000
HEADS = 4
DHEAD = 32
NBLK = 625
E_TOT = N_EDGES + 2 * int(N_EDGES * 0.1) + N_NODES

NPAD = 10240
NW = 32
C = 128
GROUPS = C // 16
CPW = 98
EPAD = NW * CPW * C
TOT_CHUNKS = NW * CPW
INV_SQRT_DH = 0.17677669529663687

_P = np.zeros((NBLK, NBLK), np.float32)
_j = np.arange(NBLK)
_P[_j, (2 * _j) % NBLK] += 1.0
_P[_j, (2 * _j + 1) % NBLK] += 1.0


def _dot(a, b):
    return jax.lax.dot(a, b, precision=jax.lax.Precision.HIGHEST,
                       preferred_element_type=jnp.float32)


def _dense_body(eview_ref, distsT_ref, s_ref, wh1_ref, wh2_ref, wq_ref,
                wk_ref, bh_ref, perm_ref, qa_ref, ka_ref, qb_ref, kb_ref):
    ev = eview_ref[...]
    h = ev[:, 0:LATDIM]
    for a in range(1, 16):
        h = h + ev[:, a * LATDIM:(a + 1) * LATDIM]
    t = _dot(s_ref[...], wh1_ref[...])
    p1 = _dot(distsT_ref[...], t)
    g = _dot(_dot(perm_ref[...], h), wh2_ref[...])
    pos_b = g * (1.0 / ANCHOR) + bh_ref[...]
    pos_a = p1 * (1.0 / ANCHOR)
    qa_ref[...] = _dot(pos_a, wq_ref[...])
    ka_ref[...] = _dot(pos_a, wk_ref[...])
    qb_ref[...] = _dot(pos_b, wq_ref[...])
    kb_ref[...] = _dot(pos_b, wk_ref[...])


_dense_call = pl.pallas_call(
    _dense_body,
    out_shape=[
        jax.ShapeDtypeStruct((N_NODES, LATDIM), jnp.float32),
        jax.ShapeDtypeStruct((N_NODES, LATDIM), jnp.float32),
        jax.ShapeDtypeStruct((NBLK, LATDIM), jnp.float32),
        jax.ShapeDtypeStruct((NBLK, LATDIM), jnp.float32),
    ],
)

_sc_mesh = plsc.VectorSubcoreMesh(core_axis_name="c", subcore_axis_name="s")


@functools.partial(
    pl.kernel, mesh=_sc_mesh,
    out_type=[
        jax.ShapeDtypeStruct((TOT_CHUNKS, HEADS, C), jnp.float32),
        jax.ShapeDtypeStruct((2 * HEADS, NPAD), jnp.float32),
    ],
    scratch_types=[
        pltpu.VMEM((2, C), jnp.int32),
        pltpu.VMEM((2, C), jnp.int32),
        pltpu.VMEM((C, LATDIM), jnp.float32),
        pltpu.VMEM((C, LATDIM), jnp.float32),
        pltpu.VMEM((C, LATDIM), jnp.float32),
        pltpu.VMEM((C, LATDIM), jnp.float32),
        pltpu.VMEM((HEADS, C), jnp.float32),
        pltpu.VMEM((2048,), jnp.float32),
        pltpu.VMEM_SHARED((NPAD,), jnp.float32),
        pltpu.VMEM_SHARED((NPAD,), jnp.float32),
        pltpu.VMEM_SHARED((NPAD,), jnp.float32),
        pltpu.VMEM_SHARED((NPAD,), jnp.float32),
        pltpu.SemaphoreType.DMA,
        pltpu.SemaphoreType.DMA,
        pltpu.SemaphoreType.DMA,
        pltpu.SemaphoreType.DMA,
    ],
    compiler_params=pltpu.CompilerParams(needs_layout_passes=False),
)
def _edge_kernel(q_hbm, k_hbm, rc_hbm, expl_hbm, den_hbm,
                 rcb0, rcb1, q0, k0, q1, k1, explb, zbuf,
                 den0, den1, den2, den3, sq0, sk0, sq1, sk1):
    cid = lax.axis_index("c")
    sid = lax.axis_index("s")
    wid = sid * 2 + cid
    dens = [den0, den1, den2, den3]
    bufs = [(rcb0, q0, k0, sq0, sk0), (rcb1, q1, k1, sq1, sk1)]

    @pl.when(sid == 0)
    def _init():
        def zb(i, carry):
            zbuf[pl.ds(i * 16, 16)] = jnp.zeros((16,), jnp.float32)
            return carry
        lax.fori_loop(0, 128, zb, 0)
        for dh in dens:
            for tt in range(5):
                pltpu.sync_copy(zbuf, dh.at[pl.ds(tt * 2048, 2048)])

    plsc.subcore_barrier()
    base = wid * CPW

    def issue(c, b):
        rcb, qb, kb, sq, sk = bufs[b]
        pltpu.sync_copy(rc_hbm.at[c], rcb)
        pltpu.async_copy(q_hbm.at[rcb.at[0]], qb, sq)
        pltpu.async_copy(k_hbm.at[rcb.at[1]], kb, sk)

    def wait_rows(b):
        rcb, qb, kb, sq, sk = bufs[b]
        pltpu.make_async_copy(q_hbm.at[rcb.at[0]], qb, sq).wait()
        pltpu.make_async_copy(k_hbm.at[rcb.at[1]], kb, sk).wait()

    def compute(c, b):
        rcb, qb, kb, sq, sk = bufs[b]
        ebase = c * C

        @plsc.parallel_loop(0, GROUPS, 1)
        def group_body(g):
            lbase = g * 16
            ev = lax.broadcasted_iota(jnp.int32, (16,), 0) + lbase
            accs = [jnp.zeros((16,), jnp.float32) for _ in range(HEADS)]
            for dd in range(LATDIM):
                dv = jnp.full((16,), dd, jnp.int32)
                qv = plsc.load_gather(qb, [ev, dv])
                kv = plsc.load_gather(kb, [ev, dv])
                accs[dd // DHEAD] = accs[dd // DHEAD] + qv * kv
            gid = ev + ebase
            maskf = jnp.where(gid < E_TOT, 1.0, 0.0).astype(jnp.float32)
            for hh in range(HEADS):
                lg = accs[hh] * INV_SQRT_DH
                lg = jnp.minimum(jnp.maximum(lg, -10.0), 10.0)
                explb[hh, pl.ds(lbase, 16)] = jnp.exp(lg) * maskf

        pltpu.sync_copy(explb, expl_hbm.at[c])
        for hh in range(HEADS):
            pltpu.sync_copy(explb.at[hh], dens[hh].at[rcb.at[0]], add=True)

    issue(base, 0)

    def body2(i, carry):
        c0 = base + 2 * i
        issue(c0 + 1, 1)
        wait_rows(0)
        compute(c0, 0)

        @pl.when(i < CPW // 2 - 1)
        def _next():
            issue(c0 + 2, 0)

        wait_rows(1)
        compute(c0 + 1, 1)
        return carry

    lax.fori_loop(0, CPW // 2, body2, 0)
    plsc.subcore_barrier()

    @pl.when(sid == 0)
    def _flush():
        for hh in range(HEADS):
            pltpu.sync_copy(dens[hh], den_hbm.at[cid * HEADS + hh])


@functools.partial(
    pl.kernel, mesh=_sc_mesh,
    out_type=jax.ShapeDtypeStruct((EPAD,), jnp.float32),
    scratch_types=[
        pltpu.VMEM((C,), jnp.int32),
        pltpu.VMEM((HEADS, C), jnp.float32),
        pltpu.VMEM((C,), jnp.float32),
        pltpu.VMEM((C,), jnp.float32),
        pltpu.VMEM((640,), jnp.float32),
        pltpu.VMEM((640,), jnp.float32),
        pltpu.VMEM_SHARED((NPAD,), jnp.float32),
        pltpu.VMEM_SHARED((NPAD,), jnp.float32),
        pltpu.VMEM_SHARED((NPAD,), jnp.float32),
        pltpu.VMEM_SHARED((NPAD,), jnp.float32),
        pltpu.SemaphoreType.DMA,
    ],
    compiler_params=pltpu.CompilerParams(needs_layout_passes=False),
)
def _norm_kernel(expl_hbm, rows_hbm, den_hbm, att_hbm,
                 rowidx, explb, db, attb, sa, sb,
                 sd0, sd1, sd2, sd3, sem0):
    cid = lax.axis_index("c")
    sid = lax.axis_index("s")
    wid = sid * 2 + cid
    sds = [sd0, sd1, sd2, sd3]
    quarter = sid // HEADS

    for hh in range(HEADS):
        @pl.when(sid % HEADS == hh)
        def _build(hh=hh):
            def bchunk(t, carry):
                off = quarter * 2560 + t * 640
                pltpu.sync_copy(den_hbm.at[hh, pl.ds(off, 640)], sa)
                pltpu.sync_copy(den_hbm.at[HEADS + hh, pl.ds(off, 640)], sb)

                def vec(i, c2):
                    sl = pl.ds(i * 16, 16)
                    sa[sl] = sa[sl] + sb[sl] + 1e-8
                    return c2

                lax.fori_loop(0, 40, vec, 0)
                pltpu.sync_copy(sa, sds[hh].at[pl.ds(off, 640)])
                return carry

            lax.fori_loop(0, 4, bchunk, 0)

    plsc.subcore_barrier()

    def chunk_body(c, carry):
        gchunk = wid * CPW + c
        ebase = gchunk * C
        pltpu.sync_copy(rows_hbm.at[pl.ds(ebase, C)], rowidx)
        pltpu.sync_copy(expl_hbm.at[gchunk], explb)
        for hh in range(HEADS):
            pltpu.async_copy(sds[hh].at[rowidx], db, sem0).wait()

            def group_body(g, gcarry, hh=hh):
                sl = pl.ds(g * 16, 16)
                e = explb[hh, sl]
                if hh == 0:
                    attb[sl] = e / db[sl]
                else:
                    attb[sl] = attb[sl] + e / db[sl]
                return gcarry

            lax.fori_loop(0, GROUPS, group_body, 0)
        pltpu.sync_copy(attb, att_hbm.at[pl.ds(ebase, C)])
        return carry

    lax.fori_loop(0, CPW, chunk_body, 0)


def kernel(embeds, dists_array, anchorset_id, edge_index, Wh, bh, Wq, Wk, Wv):
    embeds = embeds.astype(jnp.float32)
    eview = embeds.reshape(NBLK, 16 * LATDIM)
    dists_t = dists_array.T.astype(jnp.float32)
    s = embeds[anchorset_id]
    wh1 = Wh[:LATDIM]
    wh2 = Wh[LATDIM:]
    bh_row = bh.reshape(1, LATDIM)

    qa, ka, qb, kb = _dense_call(eview, dists_t, s, wh1, wh2, Wq, Wk,
                                 bh_row, jnp.asarray(_P))
    q = qa + jnp.tile(qb, (16, 1))
    k = ka + jnp.tile(kb, (16, 1))

    rows = edge_index[0]
    cols = edge_index[1]
    n_add = int(N_EDGES * 0.1)
    akey = jax.random.key(42)
    k1, k2 = jax.random.split(akey)
    idx1 = jax.random.randint(k1, (n_add,), 0, N_EDGES)
    idx2 = jax.random.randint(k2, (n_add,), 0, N_EDGES)
    add_rows = rows[idx1]
    add_cols = cols[idx2]
    loops = jnp.arange(N_NODES, dtype=rows.dtype)
    new_rows = jnp.concatenate([add_rows, add_cols, loops, rows])
    new_cols = jnp.concatenate([add_cols, add_rows, loops, cols])

    pad = jnp.zeros((EPAD - E_TOT,), new_rows.dtype)
    rpad = jnp.concatenate([new_rows, pad]).astype(jnp.int32)
    cpad = jnp.concatenate([new_cols, pad]).astype(jnp.int32)
    rc = jnp.stack([rpad.reshape(TOT_CHUNKS, C),
                    cpad.reshape(TOT_CHUNKS, C)], axis=1)

    expl, den = _edge_kernel(q, k, rc)
    att_pad = _norm_kernel(expl, rpad, den)
    att_edge = att_pad[:E_TOT]
    new_edge_index = jnp.stack([new_rows, new_cols])
    return att_edge, new_edge_index


# speedup vs baseline: 5.2375x; 1.0037x over previous
import functools

import jax
import jax.numpy as jnp
import numpy as np
from jax import lax
from jax.experimental import pallas as pl
from jax.experimental.pallas import tpu as pltpu
from jax.experimental.pallas import tpu_sc as plsc

LATDIM = 128
ANCHOR = 32
N_NODES = 10000
N_EDGES = 320000
HEADS = 4
DHEAD = 32
NBLK = 625
E_TOT = N_EDGES + 2 * int(N_EDGES * 0.1) + N_NODES

NPAD = 10240
NW = 32
C = 128
GROUPS = C // 16
CPW = 98
EPAD = NW * CPW * C
TOT_CHUNKS = NW * CPW
INV_SQRT_DH = 0.17677669529663687

_P = np.zeros((NBLK, NBLK), np.float32)
_j = np.arange(NBLK)
_P[_j, (2 * _j) % NBLK] += 1.0
_P[_j, (2 * _j + 1) % NBLK] += 1.0


def _dot(a, b):
    return jax.lax.dot(a, b, precision=jax.lax.Precision.HIGHEST,
                       preferred_element_type=jnp.float32)


def _dense_body(eview_ref, distsT_ref, s_ref, wh1_ref, wh2_ref, wq_ref,
                wk_ref, bh_ref, perm_ref, qa_ref, ka_ref, qb_ref, kb_ref):
    ev = eview_ref[...]
    h = ev[:, 0:LATDIM]
    for a in range(1, 16):
        h = h + ev[:, a * LATDIM:(a + 1) * LATDIM]
    t = _dot(s_ref[...], wh1_ref[...])
    p1 = _dot(distsT_ref[...], t)
    g = _dot(_dot(perm_ref[...], h), wh2_ref[...])
    pos_b = g * (1.0 / ANCHOR) + bh_ref[...]
    pos_a = p1 * (1.0 / ANCHOR)
    qa_ref[...] = _dot(pos_a, wq_ref[...])
    ka_ref[...] = _dot(pos_a, wk_ref[...])
    qb_ref[...] = _dot(pos_b, wq_ref[...])
    kb_ref[...] = _dot(pos_b, wk_ref[...])


_dense_call = pl.pallas_call(
    _dense_body,
    out_shape=[
        jax.ShapeDtypeStruct((N_NODES, LATDIM), jnp.float32),
        jax.ShapeDtypeStruct((N_NODES, LATDIM), jnp.float32),
        jax.ShapeDtypeStruct((NBLK, LATDIM), jnp.float32),
        jax.ShapeDtypeStruct((NBLK, LATDIM), jnp.float32),
    ],
)

_sc_mesh = plsc.VectorSubcoreMesh(core_axis_name="c", subcore_axis_name="s")


@functools.partial(
    pl.kernel, mesh=_sc_mesh,
    out_type=[
        jax.ShapeDtypeStruct((TOT_CHUNKS, HEADS, C), jnp.float32),
        jax.ShapeDtypeStruct((2 * HEADS, NPAD), jnp.float32),
    ],
    scratch_types=[
        pltpu.VMEM((2, C), jnp.int32),
        pltpu.VMEM((2, C), jnp.int32),
        pltpu.VMEM((C, LATDIM), jnp.float32),
        pltpu.VMEM((C, LATDIM), jnp.float32),
        pltpu.VMEM((C, LATDIM), jnp.float32),
        pltpu.VMEM((C, LATDIM), jnp.float32),
        pltpu.VMEM((HEADS, C), jnp.float32),
        pltpu.VMEM((HEADS, C), jnp.float32),
        pltpu.VMEM((C,), jnp.int32),
        pltpu.VMEM((C,), jnp.int32),
        pltpu.VMEM((2048,), jnp.float32),
        pltpu.VMEM_SHARED((NPAD,), jnp.float32),
        pltpu.VMEM_SHARED((NPAD,), jnp.float32),
        pltpu.VMEM_SHARED((NPAD,), jnp.float32),
        pltpu.VMEM_SHARED((NPAD,), jnp.float32),
        pltpu.SemaphoreType.DMA,
        pltpu.SemaphoreType.DMA,
        pltpu.SemaphoreType.DMA,
        pltpu.SemaphoreType.DMA,
        pltpu.SemaphoreType.DMA,
        pltpu.SemaphoreType.DMA,
    ],
    compiler_params=pltpu.CompilerParams(needs_layout_passes=False),
)
def _edge_kernel(q_hbm, k_hbm, rc_hbm, expl_hbm, den_hbm,
                 rcb0, rcb1, q0, k0, q1, k1, explb0, explb1, rsb0, rsb1,
                 zbuf, den0, den1, den2, den3, sq0, sk0, sq1, sk1, so0, so1):
    cid = lax.axis_index("c")
    sid = lax.axis_index("s")
    wid = sid * 2 + cid
    dens = [den0, den1, den2, den3]
    bufs = [(rcb0, q0, k0, explb0, rsb0, sq0, sk0, so0),
            (rcb1, q1, k1, explb1, rsb1, sq1, sk1, so1)]

    @pl.when(sid == 0)
    def _init():
        def zb(i, carry):
            zbuf[pl.ds(i * 16, 16)] = jnp.zeros((16,), jnp.float32)
            return carry
        lax.fori_loop(0, 128, zb, 0)
        for dh in dens:
            for tt in range(5):
                pltpu.sync_copy(zbuf, dh.at[pl.ds(tt * 2048, 2048)])

    plsc.subcore_barrier()
    base = wid * CPW

    def issue(c, b):
        rcb, qb, kb, eb, rsb, sq, sk, so = bufs[b]
        pltpu.sync_copy(rc_hbm.at[c], rcb)
        pltpu.async_copy(q_hbm.at[rcb.at[0]], qb, sq)
        pltpu.async_copy(k_hbm.at[rcb.at[1]], kb, sk)

    def wait_rows(b):
        rcb, qb, kb, eb, rsb, sq, sk, so = bufs[b]
        pltpu.make_async_copy(q_hbm.at[rcb.at[0]], qb, sq).wait()
        pltpu.make_async_copy(k_hbm.at[rcb.at[1]], kb, sk).wait()

    def drain_out(c, b):
        rcb, qb, kb, eb, rsb, sq, sk, so = bufs[b]
        pltpu.make_async_copy(eb, expl_hbm.at[c], so).wait()

    def compute(c, b):
        rcb, qb, kb, eb, rsb, sq, sk, so = bufs[b]
        ebase = c * C

        def sidx(g, carry):
            sl = pl.ds(g * 16, 16)
            rsb[sl] = rcb[0, sl]
            return carry

        lax.fori_loop(0, GROUPS, sidx, 0)

        @plsc.parallel_loop(0, GROUPS, 1)
        def group_body(g):
            lbase = g * 16
            ev = lax.broadcasted_iota(jnp.int32, (16,), 0) + lbase
            accs = [jnp.zeros((16,), jnp.float32) for _ in range(HEADS)]
            for dd in range(LATDIM):
                dv = jnp.full((16,), dd, jnp.int32)
                qv = plsc.load_gather(qb, [ev, dv])
                kv = plsc.load_gather(kb, [ev, dv])
                accs[dd // DHEAD] = accs[dd // DHEAD] + qv * kv
            gid = ev + ebase
            maskf = jnp.where(gid < E_TOT, 1.0, 0.0).astype(jnp.float32)
            for hh in range(HEADS):
                lg = accs[hh] * INV_SQRT_DH
                lg = jnp.minimum(jnp.maximum(lg, -10.0), 10.0)
                eb[hh, pl.ds(lbase, 16)] = jnp.exp(lg) * maskf

        pltpu.async_copy(eb, expl_hbm.at[c], so)
        for hh in range(HEADS):
            pltpu.sync_copy(eb.at[hh], dens[hh].at[rsb], add=True)

    def step(i, c, b):
        issue(c + 1, 1 - b)
        wait_rows(b)

        @pl.when(i > 0)
        def _drain():
            drain_out(c - 2, b)

        compute(c, b)

    issue(base, 0)

    def body2(i, carry):
        c0 = base + 2 * i
        step(i, c0, 0)
        step(i, c0 + 1, 1)
        return carry

    lax.fori_loop(0, CPW // 2 - 1, body2, 0)
    clast = base + CPW - 2
    issue(clast + 1, 1)
    wait_rows(0)
    drain_out(clast - 2, 0)
    compute(clast, 0)
    wait_rows(1)
    drain_out(clast - 1, 1)
    compute(clast + 1, 1)
    drain_out(clast, 0)
    drain_out(clast + 1, 1)
    plsc.subcore_barrier()

    @pl.when(sid == 0)
    def _flush():
        for hh in range(HEADS):
            pltpu.sync_copy(dens[hh], den_hbm.at[cid * HEADS + hh])


@functools.partial(
    pl.kernel, mesh=_sc_mesh,
    out_type=jax.ShapeDtypeStruct((EPAD,), jnp.float32),
    scratch_types=[
        pltpu.VMEM((C,), jnp.int32),
        pltpu.VMEM((HEADS, C), jnp.float32),
        pltpu.VMEM((C,), jnp.float32),
        pltpu.VMEM((C,), jnp.float32),
        pltpu.VMEM((640,), jnp.float32),
        pltpu.VMEM((640,), jnp.float32),
        pltpu.VMEM_SHARED((NPAD,), jnp.float32),
        pltpu.VMEM_SHARED((NPAD,), jnp.float32),
        pltpu.VMEM_SHARED((NPAD,), jnp.float32),
        pltpu.VMEM_SHARED((NPAD,), jnp.float32),
        pltpu.SemaphoreType.DMA,
    ],
    compiler_params=pltpu.CompilerParams(needs_layout_passes=False),
)
def _norm_kernel(expl_hbm, rows_hbm, den_hbm, att_hbm,
                 rowidx, explb, db, attb, sa, sb,
                 sd0, sd1, sd2, sd3, sem0):
    cid = lax.axis_index("c")
    sid = lax.axis_index("s")
    wid = sid * 2 + cid
    sds = [sd0, sd1, sd2, sd3]
    quarter = sid // HEADS

    for hh in range(HEADS):
        @pl.when(sid % HEADS == hh)
        def _build(hh=hh):
            def bchunk(t, carry):
                off = quarter * 2560 + t * 640
                pltpu.sync_copy(den_hbm.at[hh, pl.ds(off, 640)], sa)
                pltpu.sync_copy(den_hbm.at[HEADS + hh, pl.ds(off, 640)], sb)

                def vec(i, c2):
                    sl = pl.ds(i * 16, 16)
                    sa[sl] = sa[sl] + sb[sl] + 1e-8
                    return c2

                lax.fori_loop(0, 40, vec, 0)
                pltpu.sync_copy(sa, sds[hh].at[pl.ds(off, 640)])
                return carry

            lax.fori_loop(0, 4, bchunk, 0)

    plsc.subcore_barrier()

    def chunk_body(c, carry):
        gchunk = wid * CPW + c
        ebase = gchunk * C
        pltpu.sync_copy(rows_hbm.at[pl.ds(ebase, C)], rowidx)
        pltpu.sync_copy(expl_hbm.at[gchunk], explb)
        for hh in range(HEADS):
            pltpu.async_copy(sds[hh].at[rowidx], db, sem0).wait()

            def group_body(g, gcarry, hh=hh):
                sl = pl.ds(g * 16, 16)
                e = explb[hh, sl]
                if hh == 0:
                    attb[sl] = e / db[sl]
                else:
                    attb[sl] = attb[sl] + e / db[sl]
                return gcarry

            lax.fori_loop(0, GROUPS, group_body, 0)
        pltpu.sync_copy(attb, att_hbm.at[pl.ds(ebase, C)])
        return carry

    lax.fori_loop(0, CPW, chunk_body, 0)


def kernel(embeds, dists_array, anchorset_id, edge_index, Wh, bh, Wq, Wk, Wv):
    embeds = embeds.astype(jnp.float32)
    eview = embeds.reshape(NBLK, 16 * LATDIM)
    dists_t = dists_array.T.astype(jnp.float32)
    s = embeds[anchorset_id]
    wh1 = Wh[:LATDIM]
    wh2 = Wh[LATDIM:]
    bh_row = bh.reshape(1, LATDIM)

    qa, ka, qb, kb = _dense_call(eview, dists_t, s, wh1, wh2, Wq, Wk,
                                 bh_row, jnp.asarray(_P))
    q = qa + jnp.tile(qb, (16, 1))
    k = ka + jnp.tile(kb, (16, 1))

    rows = edge_index[0]
    cols = edge_index[1]
    n_add = int(N_EDGES * 0.1)
    akey = jax.random.key(42)
    k1, k2 = jax.random.split(akey)
    idx1 = jax.random.randint(k1, (n_add,), 0, N_EDGES)
    idx2 = jax.random.randint(k2, (n_add,), 0, N_EDGES)
    add_rows = rows[idx1]
    add_cols = cols[idx2]
    loops = jnp.arange(N_NODES, dtype=rows.dtype)
    new_rows = jnp.concatenate([add_rows, add_cols, loops, rows])
    new_cols = jnp.concatenate([add_cols, add_rows, loops, cols])

    pad = jnp.zeros((EPAD - E_TOT,), new_rows.dtype)
    rpad = jnp.concatenate([new_rows, pad]).astype(jnp.int32)
    cpad = jnp.concatenate([new_cols, pad]).astype(jnp.int32)
    rc = jnp.stack([rpad.reshape(TOT_CHUNKS, C),
                    cpad.reshape(TOT_CHUNKS, C)], axis=1)

    expl, den = _edge_kernel(q, k, rc)
    att_pad = _norm_kernel(expl, rpad, den)
    att_edge = att_pad[:E_TOT]
    new_edge_index = jnp.stack([new_rows, new_cols])
    return att_edge, new_edge_index


# speedup vs baseline: 6.1063x; 1.1659x over previous
import functools

import jax
import jax.numpy as jnp
import numpy as np
from jax import lax
from jax.experimental import pallas as pl
from jax.experimental.pallas import tpu as pltpu
from jax.experimental.pallas import tpu_sc as plsc

LATDIM = 128
ANCHOR = 32
N_NODES = 10000
N_EDGES = 320000
HEADS = 4
DHEAD = 32
NBLK = 625
E_TOT = N_EDGES + 2 * int(N_EDGES * 0.1) + N_NODES

NPAD = 10240
NW = 32
C = 128
GROUPS = C // 16
CPW = 98
EPAD = NW * CPW * C
TOT_CHUNKS = NW * CPW
INV_SQRT_DH = 0.17677669529663687

_P = np.zeros((NBLK, NBLK), np.float32)
_j = np.arange(NBLK)
_P[_j, (2 * _j) % NBLK] += 1.0
_P[_j, (2 * _j + 1) % NBLK] += 1.0


def _dot(a, b):
    return jax.lax.dot(a, b, precision=jax.lax.Precision.HIGHEST,
                       preferred_element_type=jnp.float32)


def _dense_body(eview_ref, distsT_ref, s_ref, wh1_ref, wh2_ref, wq_ref,
                wk_ref, bh_ref, perm_ref, qa_ref, ka_ref, qb_ref, kb_ref):
    ev = eview_ref[...]
    h = ev[:, 0:LATDIM]
    for a in range(1, 16):
        h = h + ev[:, a * LATDIM:(a + 1) * LATDIM]
    t = _dot(s_ref[...], wh1_ref[...])
    p1 = _dot(distsT_ref[...], t)
    g = _dot(_dot(perm_ref[...], h), wh2_ref[...])
    pos_b = g * (1.0 / ANCHOR) + bh_ref[...]
    pos_a = p1 * (1.0 / ANCHOR)
    qa_ref[...] = _dot(pos_a, wq_ref[...])
    ka_ref[...] = _dot(pos_a, wk_ref[...])
    qb_ref[...] = _dot(pos_b, wq_ref[...])
    kb_ref[...] = _dot(pos_b, wk_ref[...])


_dense_call = pl.pallas_call(
    _dense_body,
    out_shape=[
        jax.ShapeDtypeStruct((N_NODES, LATDIM), jnp.float32),
        jax.ShapeDtypeStruct((N_NODES, LATDIM), jnp.float32),
        jax.ShapeDtypeStruct((NBLK, LATDIM), jnp.float32),
        jax.ShapeDtypeStruct((NBLK, LATDIM), jnp.float32),
    ],
)

_sc_mesh = plsc.VectorSubcoreMesh(core_axis_name="c", subcore_axis_name="s")


@functools.partial(
    pl.kernel, mesh=_sc_mesh,
    out_type=[
        jax.ShapeDtypeStruct((TOT_CHUNKS, HEADS, C), jnp.float32),
        jax.ShapeDtypeStruct((2 * HEADS, NPAD), jnp.float32),
    ],
    scratch_types=[
        pltpu.VMEM((2, C), jnp.int32),
        pltpu.VMEM((2, C), jnp.int32),
        pltpu.VMEM((C, LATDIM), jnp.float32),
        pltpu.VMEM((C, LATDIM), jnp.float32),
        pltpu.VMEM((C, LATDIM), jnp.float32),
        pltpu.VMEM((C, LATDIM), jnp.float32),
        pltpu.VMEM((HEADS, C), jnp.float32),
        pltpu.VMEM((HEADS, C), jnp.float32),
        pltpu.VMEM((C,), jnp.int32),
        pltpu.VMEM((C,), jnp.int32),
        pltpu.VMEM((2048,), jnp.float32),
        pltpu.VMEM_SHARED((NPAD,), jnp.float32),
        pltpu.VMEM_SHARED((NPAD,), jnp.float32),
        pltpu.VMEM_SHARED((NPAD,), jnp.float32),
        pltpu.VMEM_SHARED((NPAD,), jnp.float32),
        pltpu.SemaphoreType.DMA,
        pltpu.SemaphoreType.DMA,
        pltpu.SemaphoreType.DMA,
        pltpu.SemaphoreType.DMA,
        pltpu.SemaphoreType.DMA,
        pltpu.SemaphoreType.DMA,
    ],
    compiler_params=pltpu.CompilerParams(needs_layout_passes=False),
)
def _edge_kernel(q_hbm, k_hbm, rc_hbm, expl_hbm, den_hbm,
                 rcb0, rcb1, q0, k0, q1, k1, explb0, explb1, rsb0, rsb1,
                 zbuf, den0, den1, den2, den3, sq0, sk0, sq1, sk1, so0, so1):
    cid = lax.axis_index("c")
    sid = lax.axis_index("s")
    wid = sid * 2 + cid
    dens = [den0, den1, den2, den3]
    bufs = [(rcb0, q0, k0, explb0, rsb0, sq0, sk0, so0),
            (rcb1, q1, k1, explb1, rsb1, sq1, sk1, so1)]

    @pl.when(sid == 0)
    def _init():
        def zb(i, carry):
            zbuf[pl.ds(i * 16, 16)] = jnp.zeros((16,), jnp.float32)
            return carry
        lax.fori_loop(0, 128, zb, 0)
        for dh in dens:
            for tt in range(5):
                pltpu.sync_copy(zbuf, dh.at[pl.ds(tt * 2048, 2048)])

    plsc.subcore_barrier()
    base = wid * CPW

    def issue(c, b):
        rcb, qb, kb, eb, rsb, sq, sk, so = bufs[b]
        pltpu.sync_copy(rc_hbm.at[c], rcb)
        pltpu.async_copy(q_hbm.at[rcb.at[0]], qb, sq)
        pltpu.async_copy(k_hbm.at[rcb.at[1]], kb, sk)

    def wait_rows(b):
        rcb, qb, kb, eb, rsb, sq, sk, so = bufs[b]
        pltpu.make_async_copy(q_hbm.at[rcb.at[0]], qb, sq).wait()
        pltpu.make_async_copy(k_hbm.at[rcb.at[1]], kb, sk).wait()

    def drain_out(c, b):
        rcb, qb, kb, eb, rsb, sq, sk, so = bufs[b]
        pltpu.make_async_copy(eb, expl_hbm.at[c], so).wait()

    def compute(c, b):
        rcb, qb, kb, eb, rsb, sq, sk, so = bufs[b]
        ebase = c * C

        def sidx(g, carry):
            sl = pl.ds(g * 16, 16)
            rsb[sl] = rcb[0, sl]
            return carry

        lax.fori_loop(0, GROUPS, sidx, 0)

        @plsc.parallel_loop(0, GROUPS, 1, unroll=2)
        def group_body(g):
            lbase = g * 16
            ev = lax.broadcasted_iota(jnp.int32, (16,), 0) + lbase
            accs = [jnp.zeros((16,), jnp.float32) for _ in range(HEADS)]
            for dd in range(LATDIM):
                dv = jnp.full((16,), dd, jnp.int32)
                qv = plsc.load_gather(qb, [ev, dv])
                kv = plsc.load_gather(kb, [ev, dv])
                accs[dd // DHEAD] = accs[dd // DHEAD] + qv * kv
            gid = ev + ebase
            maskf = jnp.where(gid < E_TOT, 1.0, 0.0).astype(jnp.float32)
            for hh in range(HEADS):
                lg = accs[hh] * INV_SQRT_DH
                lg = jnp.minimum(jnp.maximum(lg, -10.0), 10.0)
                eb[hh, pl.ds(lbase, 16)] = jnp.exp(lg) * maskf

        pltpu.async_copy(eb, expl_hbm.at[c], so)
        for hh in range(HEADS):
            pltpu.sync_copy(eb.at[hh], dens[hh].at[rsb], add=True)

    def step(i, c, b):
        issue(c + 1, 1 - b)
        wait_rows(b)

        @pl.when(i > 0)
        def _drain():
            drain_out(c - 2, b)

        compute(c, b)

    issue(base, 0)

    def body2(i, carry):
        c0 = base + 2 * i
        step(i, c0, 0)
        step(i, c0 + 1, 1)
        return carry

    lax.fori_loop(0, CPW // 2 - 1, body2, 0)
    clast = base + CPW - 2
    issue(clast + 1, 1)
    wait_rows(0)
    drain_out(clast - 2, 0)
    compute(clast, 0)
    wait_rows(1)
    drain_out(clast - 1, 1)
    compute(clast + 1, 1)
    drain_out(clast, 0)
    drain_out(clast + 1, 1)
    plsc.subcore_barrier()

    @pl.when(sid == 0)
    def _flush():
        for hh in range(HEADS):
            pltpu.sync_copy(dens[hh], den_hbm.at[cid * HEADS + hh])


@functools.partial(
    pl.kernel, mesh=_sc_mesh,
    out_type=jax.ShapeDtypeStruct((EPAD,), jnp.float32),
    scratch_types=[
        pltpu.VMEM((C,), jnp.int32),
        pltpu.VMEM((HEADS, C), jnp.float32),
        pltpu.VMEM((C,), jnp.float32),
        pltpu.VMEM((C,), jnp.float32),
        pltpu.VMEM((640,), jnp.float32),
        pltpu.VMEM((640,), jnp.float32),
        pltpu.VMEM_SHARED((NPAD,), jnp.float32),
        pltpu.VMEM_SHARED((NPAD,), jnp.float32),
        pltpu.VMEM_SHARED((NPAD,), jnp.float32),
        pltpu.VMEM_SHARED((NPAD,), jnp.float32),
        pltpu.SemaphoreType.DMA,
    ],
    compiler_params=pltpu.CompilerParams(needs_layout_passes=False),
)
def _norm_kernel(expl_hbm, rows_hbm, den_hbm, att_hbm,
                 rowidx, explb, db, attb, sa, sb,
                 sd0, sd1, sd2, sd3, sem0):
    cid = lax.axis_index("c")
    sid = lax.axis_index("s")
    wid = sid * 2 + cid
    sds = [sd0, sd1, sd2, sd3]
    quarter = sid // HEADS

    for hh in range(HEADS):
        @pl.when(sid % HEADS == hh)
        def _build(hh=hh):
            def bchunk(t, carry):
                off = quarter * 2560 + t * 640
                pltpu.sync_copy(den_hbm.at[hh, pl.ds(off, 640)], sa)
                pltpu.sync_copy(den_hbm.at[HEADS + hh, pl.ds(off, 640)], sb)

                def vec(i, c2):
                    sl = pl.ds(i * 16, 16)
                    sa[sl] = sa[sl] + sb[sl] + 1e-8
                    return c2

                lax.fori_loop(0, 40, vec, 0)
                pltpu.sync_copy(sa, sds[hh].at[pl.ds(off, 640)])
                return carry

            lax.fori_loop(0, 4, bchunk, 0)

    plsc.subcore_barrier()

    def chunk_body(c, carry):
        gchunk = wid * CPW + c
        ebase = gchunk * C
        pltpu.sync_copy(rows_hbm.at[pl.ds(ebase, C)], rowidx)
        pltpu.sync_copy(expl_hbm.at[gchunk], explb)
        for hh in range(HEADS):
            pltpu.async_copy(sds[hh].at[rowidx], db, sem0).wait()

            def group_body(g, gcarry, hh=hh):
                sl = pl.ds(g * 16, 16)
                e = explb[hh, sl]
                if hh == 0:
                    attb[sl] = e / db[sl]
                else:
                    attb[sl] = attb[sl] + e / db[sl]
                return gcarry

            lax.fori_loop(0, GROUPS, group_body, 0)
        pltpu.sync_copy(attb, att_hbm.at[pl.ds(ebase, C)])
        return carry

    lax.fori_loop(0, CPW, chunk_body, 0)


def kernel(embeds, dists_array, anchorset_id, edge_index, Wh, bh, Wq, Wk, Wv):
    embeds = embeds.astype(jnp.float32)
    eview = embeds.reshape(NBLK, 16 * LATDIM)
    dists_t = dists_array.T.astype(jnp.float32)
    s = embeds[anchorset_id]
    wh1 = Wh[:LATDIM]
    wh2 = Wh[LATDIM:]
    bh_row = bh.reshape(1, LATDIM)

    qa, ka, qb, kb = _dense_call(eview, dists_t, s, wh1, wh2, Wq, Wk,
                                 bh_row, jnp.asarray(_P))
    q = qa + jnp.tile(qb, (16, 1))
    k = ka + jnp.tile(kb, (16, 1))

    rows = edge_index[0]
    cols = edge_index[1]
    n_add = int(N_EDGES * 0.1)
    akey = jax.random.key(42)
    k1, k2 = jax.random.split(akey)
    idx1 = jax.random.randint(k1, (n_add,), 0, N_EDGES)
    idx2 = jax.random.randint(k2, (n_add,), 0, N_EDGES)
    add_rows = rows[idx1]
    add_cols = cols[idx2]
    loops = jnp.arange(N_NODES, dtype=rows.dtype)
    new_rows = jnp.concatenate([add_rows, add_cols, loops, rows])
    new_cols = jnp.concatenate([add_cols, add_rows, loops, cols])

    pad = jnp.zeros((EPAD - E_TOT,), new_rows.dtype)
    rpad = jnp.concatenate([new_rows, pad]).astype(jnp.int32)
    cpad = jnp.concatenate([new_cols, pad]).astype(jnp.int32)
    rc = jnp.stack([rpad.reshape(TOT_CHUNKS, C),
                    cpad.reshape(TOT_CHUNKS, C)], axis=1)

    expl, den = _edge_kernel(q, k, rc)
    att_pad = _norm_kernel(expl, rpad, den)
    att_edge = att_pad[:E_TOT]
    new_edge_index = jnp.stack([new_rows, new_cols])
    return att_edge, new_edge_index
